# Initial kernel scaffold; baseline (speedup 1.0000x reference)
import jax
import jax.numpy as jnp
from jax.experimental import pallas as pl


def kernel(x, edge_index, batch, W1, a_src1, a_dst1, b1, bn1_g, bn1_b, bn1_m, bn1_v, W2, a_src2, a_dst2, b2, bn2_g, bn2_b, bn2_m, bn2_v, W3, a_src3, a_dst3, b3, bn3_g, bn3_b, bn3_m, bn3_v):
    raise NotImplementedError("write your pallas kernel here")


# speedup vs baseline: 19.6142x; 19.6142x over previous
import functools
import jax
import jax.numpy as jnp
from jax import lax
from jax.experimental import pallas as pl
from jax.experimental.pallas import tpu as pltpu
from jax.experimental.pallas import tpu_sc as plsc

N = 50000
B = 1024
NPAD = 51200
NPOOL = 1152
EK = 512
ECH = 104
EPT = EK * ECH
E_PAD = EPT * 16
NHALF = 25600
ACCR = NHALF + 8
BLK = 512
EPS = 1e-16


def _edge_body(heads, srcr, dstr, hflat, sed, zrows, out,
               srcv, dstv, srcv2, dstv3, seds, sedd, hrows, outb, acc,
               sem0, sem1, sem2):
    cid = lax.axis_index("c")
    sid = lax.axis_index("s")
    lanes = lax.iota(jnp.int32, 16)
    zero16 = lanes * 0

    pltpu.sync_copy(zrows.at[pl.ds(0, EK)], outb)

    for p in range(NPAD // NHALF):
        nbase = p * NHALF

        @pl.when(sid == 0)
        def _():
            pltpu.sync_copy(zrows.at[pl.ds(0, ACCR)], acc)
        plsc.subcore_barrier()

        def chunk(ch, carry):
            rb = sid * (EPT // 128) + ch * (EK // 128)
            pltpu.sync_copy(srcr.at[pl.ds(rb, EK // 128)], srcv)
            pltpu.sync_copy(dstr.at[pl.ds(rb, EK // 128)], dstv)
            for i in range(EK // 128):
                for j in range(8):
                    sl16 = pl.ds(j * 16, 16)
                    srcv2[i, sl16] = srcv[i, sl16] + cid * NPAD
                    dl = dstv[i, sl16] - nbase
                    ok = (dl >= 0) & (dl < NHALF)
                    dstv3[i, sl16] = jnp.where(ok, dl, NHALF)
            cps = []
            for g in range(EK // 128):
                sl = pl.ds(g * 128, 128)
                cps.append(pltpu.async_copy(sed.at[srcv.at[g]], seds.at[sl],
                                            sem0))
                cps.append(pltpu.async_copy(sed.at[dstv.at[g]], sedd.at[sl],
                                            sem1))
                cps.append(pltpu.async_copy(hflat.at[srcv2.at[g]],
                                            hrows.at[sl], sem2))
            for cp in cps:
                cp.wait()

            def group(g, c2):
                rows = g * 16 + lanes
                if heads == 4:
                    ws = []
                    for h in range(4):
                        es = plsc.load_gather(seds, [rows, zero16 + h])
                        ed = plsc.load_gather(sedd, [rows, zero16 + 4 + h])
                        e = es + ed
                        e = jnp.where(e >= 0, e, e * jnp.float32(0.2))
                        ws.append(jnp.exp(e))
                    c0 = cid == 0
                    wlo = jnp.where(c0, ws[0], ws[2])
                    whi = jnp.where(c0, ws[1], ws[3])
                    for h in range(4):
                        plsc.store_scatter(outb, [rows, zero16 + 32 + h],
                                           ws[h])
                    for c in range(32):
                        col = plsc.load_gather(hrows, [rows, zero16 + c])
                        w = wlo if c < 16 else whi
                        plsc.store_scatter(outb, [rows, zero16 + c], col * w)
                else:
                    es = plsc.load_gather(seds, [rows, zero16])
                    ed = plsc.load_gather(sedd, [rows, zero16 + 1])
                    e = es + ed
                    e = jnp.where(e >= 0, e, e * jnp.float32(0.2))
                    w = jnp.exp(e)
                    plsc.store_scatter(outb, [rows, zero16 + 32], w)
                    for c in range(32):
                        col = plsc.load_gather(hrows, [rows, zero16 + c])
                        plsc.store_scatter(outb, [rows, zero16 + c], col * w)
                return c2
            lax.fori_loop(0, EK // 16, group, 0)

            for g in range(EK // 128):
                pltpu.sync_copy(outb.at[pl.ds(g * 128, 128)],
                                acc.at[dstv3.at[g]], add=True)
            return carry
        lax.fori_loop(0, ECH, chunk, 0)

        plsc.subcore_barrier()
        rows_per = NHALF // 16
        pltpu.sync_copy(
            acc.at[pl.ds(sid * rows_per, rows_per)],
            out.at[cid, pl.ds(nbase + sid * rows_per, rows_per)])
        plsc.subcore_barrier()


def _make_edge_kernel(heads):
    mesh = plsc.VectorSubcoreMesh(core_axis_name="c", subcore_axis_name="s", num_cores=2, num_subcores=16)
    return functools.partial(
        pl.kernel,
        out_type=jax.ShapeDtypeStruct((2, NPAD, 40), jnp.float32),
        mesh=mesh,
        compiler_params=pltpu.CompilerParams(needs_layout_passes=False, use_tc_tiling_on_sc=False),
        scratch_types=[
            pltpu.VMEM((EK // 128, 128), jnp.int32),
            pltpu.VMEM((EK // 128, 128), jnp.int32),
            pltpu.VMEM((EK // 128, 128), jnp.int32),
            pltpu.VMEM((EK // 128, 128), jnp.int32),
            pltpu.VMEM((EK, 8), jnp.float32),
            pltpu.VMEM((EK, 8), jnp.float32),
            pltpu.VMEM((EK, 32), jnp.float32),
            pltpu.VMEM((EK, 40), jnp.float32),
            pltpu.VMEM_SHARED((ACCR, 40), jnp.float32),
            pltpu.SemaphoreType.DMA,
            pltpu.SemaphoreType.DMA,
            pltpu.SemaphoreType.DMA,
        ],
    )(functools.partial(_edge_body, heads))


def _pool_body(pflat, bidx, zrows, out, rowsb, biv, acc):
    cid = lax.axis_index("c")
    sid = lax.axis_index("s")

    @pl.when(sid == 0)
    def _():
        pltpu.sync_copy(zrows.at[pl.ds(0, NPOOL)], acc)
    plsc.subcore_barrier()

    def chunk(ch, carry):
        rbase = sid * 3200 + ch * 640
        pltpu.sync_copy(pflat.at[pl.ds(cid * NPAD + rbase, 640)], rowsb)
        pltpu.sync_copy(bidx.at[pl.ds(sid * 25 + ch * 5, 5)], biv)
        for g in range(5):
            pltpu.sync_copy(rowsb.at[pl.ds(g * 128, 128)],
                            acc.at[biv.at[g]], add=True)
        return carry
    lax.fori_loop(0, 5, chunk, 0)

    plsc.subcore_barrier()
    sl = pl.ds(sid * (NPOOL // 16), NPOOL // 16)
    pltpu.sync_copy(acc.at[sl], out.at[cid, sl])


def _make_pool_kernel():
    return functools.partial(
        pl.kernel,
        out_type=jax.ShapeDtypeStruct((2, NPOOL, 40), jnp.float32),
        mesh=plsc.VectorSubcoreMesh(core_axis_name="c", subcore_axis_name="s",
                                    num_cores=2, num_subcores=16),
        compiler_params=pltpu.CompilerParams(needs_layout_passes=False, use_tc_tiling_on_sc=False),
        scratch_types=[
            pltpu.VMEM((640, 40), jnp.float32),
            pltpu.VMEM((5, 128), jnp.int32),
            pltpu.VMEM_SHARED((NPOOL, 40), jnp.float32),
        ],
    )(_pool_body)


def _tc_in_body(x_ref, w_ref, aa_ref, ha_ref, hb_ref, sed_ref):
    h = jnp.dot(x_ref[...], w_ref[...], preferred_element_type=jnp.float32)
    sed = jnp.dot(h, aa_ref[...], preferred_element_type=jnp.float32)
    ha_ref[...] = h[:, :32]
    hb_ref[...] = h[:, 32:]
    sed_ref[...] = sed


def _norm_concat(a0, a1, heads_prev):
    if heads_prev == 4:
        return jnp.concatenate([
            a0[:, 0:16] / (a0[:, 32:33] + EPS),
            a0[:, 16:32] / (a0[:, 33:34] + EPS),
            a1[:, 0:16] / (a1[:, 34:35] + EPS),
            a1[:, 16:32] / (a1[:, 35:36] + EPS),
        ], axis=1)
    return jnp.concatenate([
        a0[:, :32] / (a0[:, 32:33] + EPS),
        a1[:, :32] / (a1[:, 32:33] + EPS),
    ], axis=1)


def _epilogue(y, prm_ref, pid):
    y = y + prm_ref[0:1, :]
    y = y * prm_ref[1:2, :] + prm_ref[2:3, :]
    z = jnp.where(y > 0, y, jnp.exp(y) - 1.0)
    rid = pid * BLK + lax.broadcasted_iota(jnp.int32, (BLK, 1), 0)
    return jnp.where(rid < N, z, 0.0), rid


def _tc_mid_body(heads_prev, a0_ref, a1_ref, prm_ref, w_ref, aa_ref,
                 ha_ref, hb_ref, sed_ref):
    y = _norm_concat(a0_ref[...], a1_ref[...], heads_prev)
    z, _ = _epilogue(y, prm_ref, pl.program_id(0))
    h = jnp.dot(z, w_ref[...], preferred_element_type=jnp.float32)
    sed = jnp.dot(h, aa_ref[...], preferred_element_type=jnp.float32)
    ha_ref[...] = h[:, :32]
    hb_ref[...] = h[:, 32:]
    sed_ref[...] = sed


def _tc_fin_body(a0_ref, a1_ref, prm_ref, pa_ref, pb_ref):
    y = _norm_concat(a0_ref[...], a1_ref[...], 1)
    z, rid = _epilogue(y, prm_ref, pl.program_id(0))
    cnt = jnp.where(rid < N, 1.0, 0.0)
    pad = jnp.zeros((BLK, 7), jnp.float32)
    pa_ref[...] = jnp.concatenate([z[:, :32], cnt, pad], axis=1)
    pb_ref[...] = jnp.concatenate([z[:, 32:], cnt, pad], axis=1)


def _tc_div_body(p0_ref, p1_ref, o_ref):
    cnt = jnp.maximum(p0_ref[:, 32:33], 1.0)
    o_ref[...] = jnp.concatenate(
        [p0_ref[:, :32], p1_ref[:, :32]], axis=1) / cnt


def _row_spec(d):
    return pl.BlockSpec((BLK, d), lambda i: (i, 0))


def _full_spec(shape):
    return pl.BlockSpec(shape, lambda i: tuple(0 for _ in shape))


def _tc_in(xp, w, aa):
    return pl.pallas_call(
        _tc_in_body,
        grid=(NPAD // BLK,),
        in_specs=[_row_spec(xp.shape[1]), _full_spec(w.shape),
                  _full_spec(aa.shape)],
        out_specs=[_row_spec(32), _row_spec(32), _row_spec(8)],
        out_shape=[jax.ShapeDtypeStruct((NPAD, 32), jnp.float32),
                   jax.ShapeDtypeStruct((NPAD, 32), jnp.float32),
                   jax.ShapeDtypeStruct((NPAD, 8), jnp.float32)],
    )(xp, w, aa)


def _tc_mid(heads_prev, a0, a1, prm, w, aa):
    return pl.pallas_call(
        functools.partial(_tc_mid_body, heads_prev),
        grid=(NPAD // BLK,),
        in_specs=[_row_spec(40), _row_spec(40), _full_spec(prm.shape),
                  _full_spec(w.shape), _full_spec(aa.shape)],
        out_specs=[_row_spec(32), _row_spec(32), _row_spec(8)],
        out_shape=[jax.ShapeDtypeStruct((NPAD, 32), jnp.float32),
                   jax.ShapeDtypeStruct((NPAD, 32), jnp.float32),
                   jax.ShapeDtypeStruct((NPAD, 8), jnp.float32)],
    )(a0, a1, prm, w, aa)


def _tc_fin(a0, a1, prm):
    return pl.pallas_call(
        _tc_fin_body,
        grid=(NPAD // BLK,),
        in_specs=[_row_spec(40), _row_spec(40), _full_spec(prm.shape)],
        out_specs=[_row_spec(40), _row_spec(40)],
        out_shape=[jax.ShapeDtypeStruct((NPAD, 40), jnp.float32),
                   jax.ShapeDtypeStruct((NPAD, 40), jnp.float32)],
    )(a0, a1, prm)


def _tc_div(p0, p1):
    return pl.pallas_call(
        _tc_div_body,
        grid=(B // BLK,),
        in_specs=[_row_spec(40), _row_spec(40)],
        out_specs=_row_spec(64),
        out_shape=jax.ShapeDtypeStruct((B, 64), jnp.float32),
    )(p0, p1)


def _pack_aa(a_s, a_d):
    heads, c = a_s.shape
    out = jnp.zeros((64, 8), jnp.float32)
    if heads == 4:
        for h in range(4):
            out = out.at[h * c:(h + 1) * c, h].set(a_s[h])
            out = out.at[h * c:(h + 1) * c, 4 + h].set(a_d[h])
    else:
        out = out.at[:, 0].set(a_s[0])
        out = out.at[:, 1].set(a_d[0])
    return out


def _pack_prm(b, g, bb, m, v):
    scale = g / jnp.sqrt(v + 1e-5)
    shift = bb - m * scale
    prm = jnp.zeros((8, 64), jnp.float32)
    return prm.at[0].set(b).at[1].set(scale).at[2].set(shift)


def kernel(x, edge_index, batch, W1, a_src1, a_dst1, b1, bn1_g, bn1_b, bn1_m,
           bn1_v, W2, a_src2, a_dst2, b2, bn2_g, bn2_b, bn2_m, bn2_v, W3,
           a_src3, a_dst3, b3, bn3_g, bn3_b, bn3_m, bn3_v):
    f32 = jnp.float32
    n = x.shape[0]
    loop = jnp.arange(n, dtype=jnp.int32)
    padi = jnp.full((E_PAD - n - edge_index.shape[1],), n, jnp.int32)
    src = jnp.concatenate([edge_index[0].astype(jnp.int32), loop, padi])
    dst = jnp.concatenate([edge_index[1].astype(jnp.int32), loop, padi])
    srcr = src.reshape(E_PAD // 128, 128)
    dstr = dst.reshape(E_PAD // 128, 128)
    zrows = jnp.zeros((NPAD, 40), f32)

    xp = jnp.zeros((NPAD, 16), f32).at[:n, :x.shape[1]].set(x)
    w1p = jnp.zeros((16, 64), f32).at[:W1.shape[0]].set(W1)

    ha, hb, sed = _tc_in(xp, w1p, _pack_aa(a_src1, a_dst1))
    edge1 = _make_edge_kernel(4)
    a = edge1(srcr, dstr, jnp.concatenate([ha, hb]), sed, zrows)

    prm1 = _pack_prm(b1, bn1_g, bn1_b, bn1_m, bn1_v)
    ha, hb, sed = _tc_mid(4, a[0], a[1], prm1, W2, _pack_aa(a_src2, a_dst2))
    a = edge1(srcr, dstr, jnp.concatenate([ha, hb]), sed, zrows)

    prm2 = _pack_prm(b2, bn2_g, bn2_b, bn2_m, bn2_v)
    ha, hb, sed = _tc_mid(4, a[0], a[1], prm2, W3, _pack_aa(a_src3, a_dst3))
    edge3 = _make_edge_kernel(1)
    a = edge3(srcr, dstr, jnp.concatenate([ha, hb]), sed, zrows)

    prm3 = _pack_prm(b3, bn3_g, bn3_b, bn3_m, bn3_v)
    pa, pb = _tc_fin(a[0], a[1], prm3)

    bpad = jnp.concatenate(
        [batch.astype(jnp.int32), jnp.full((NPAD - n,), B, jnp.int32)])
    p = _make_pool_kernel()(jnp.concatenate([pa, pb]),
                     bpad.reshape(NPAD // 128, 128), zrows)
    return _tc_div(p[0, :B], p[1, :B])


# speedup vs baseline: 21.6818x; 1.1054x over previous
import functools
import jax
import jax.numpy as jnp
from jax import lax
from jax.experimental import pallas as pl
from jax.experimental.pallas import tpu as pltpu
from jax.experimental.pallas import tpu_sc as plsc

N = 50000
B = 1024
NPAD = 51200
NPOOL = 1152
EK = 256
ECH = 208
EPT = EK * ECH
E_PAD = EPT * 16
NHALF = 25600
ACCR = NHALF + 8
BLK = 512
EPS = 1e-16


def _edge_body(heads, srcr, dstr, hflat, sed, zrows, out,
               srcv, dstv, srcv2a, srcv2b, dstv3a, dstv3b,
               sedsa, sedsb, sedda, seddb, hrowsa, hrowsb, outb, acc,
               semi, sem0, sem1, sem2, sems):
    cid = lax.axis_index("c")
    sid = lax.axis_index("s")
    lanes = lax.iota(jnp.int32, 16)
    zero16 = lanes * 0
    srcv2_ = (srcv2a, srcv2b)
    dstv3_ = (dstv3a, dstv3b)
    seds_ = (sedsa, sedsb)
    sedd_ = (sedda, seddb)
    hrows_ = (hrowsa, hrowsb)

    pltpu.sync_copy(zrows.at[pl.ds(0, EK)], outb)

    for p in range(NPAD // NHALF):
        nbase = p * NHALF

        @pl.when(sid == 0)
        def _():
            pltpu.sync_copy(zrows.at[pl.ds(0, ACCR)], acc)
        plsc.subcore_barrier()

        def issue(ch, b):
            rb = sid * (EPT // 128) + ch * (EK // 128)
            c1 = pltpu.async_copy(srcr.at[pl.ds(rb, EK // 128)], srcv, semi)
            c2 = pltpu.async_copy(dstr.at[pl.ds(rb, EK // 128)], dstv, semi)
            c1.wait()
            c2.wait()
            for i in range(EK // 128):
                for j in range(8):
                    sl16 = pl.ds(j * 16, 16)
                    srcv2_[b][i, sl16] = srcv[i, sl16] + cid * NPAD
                    dl = dstv[i, sl16] - nbase
                    ok = (dl >= 0) & (dl < NHALF)
                    dstv3_[b][i, sl16] = jnp.where(ok, dl, NHALF)
            for g in range(EK // 128):
                sl = pl.ds(g * 128, 128)
                pltpu.async_copy(sed.at[srcv.at[g]], seds_[b].at[sl], sem0)
                pltpu.async_copy(sed.at[dstv.at[g]], sedd_[b].at[sl], sem1)
                pltpu.async_copy(hflat.at[srcv2_[b].at[g]],
                                 hrows_[b].at[sl], sem2)

        issue(0, 0)

        def chunk2(ii, carry):
            for b in range(2):
                ch = ii * 2 + b
                pltpu.make_async_copy(
                    sed.at[pl.ds(0, EK)], seds_[b], sem0).wait()
                pltpu.make_async_copy(
                    sed.at[pl.ds(0, EK)], sedd_[b], sem1).wait()
                pltpu.make_async_copy(
                    hflat.at[pl.ds(0, EK)], hrows_[b], sem2).wait()

                @pl.when(ch + 1 < ECH)
                def _():
                    issue(ch + 1, 1 - b)

                seds = seds_[b]
                sedd = sedd_[b]
                hrows = hrows_[b]

                def group(g, c2):
                    rows = g * 16 + lanes
                    if heads == 4:
                        ws = []
                        for h in range(4):
                            es = plsc.load_gather(seds, [rows, zero16 + h])
                            ed = plsc.load_gather(sedd, [rows, zero16 + 4 + h])
                            e = es + ed
                            e = jnp.where(e >= 0, e, e * jnp.float32(0.2))
                            ws.append(jnp.exp(e))
                        c0 = cid == 0
                        wlo = jnp.where(c0, ws[0], ws[2])
                        whi = jnp.where(c0, ws[1], ws[3])
                        for h in range(4):
                            plsc.store_scatter(outb, [rows, zero16 + 32 + h],
                                               ws[h])
                        for c in range(32):
                            col = plsc.load_gather(hrows, [rows, zero16 + c])
                            w = wlo if c < 16 else whi
                            plsc.store_scatter(outb, [rows, zero16 + c],
                                               col * w)
                    else:
                        es = plsc.load_gather(seds, [rows, zero16])
                        ed = plsc.load_gather(sedd, [rows, zero16 + 1])
                        e = es + ed
                        e = jnp.where(e >= 0, e, e * jnp.float32(0.2))
                        w = jnp.exp(e)
                        plsc.store_scatter(outb, [rows, zero16 + 32], w)
                        for c in range(32):
                            col = plsc.load_gather(hrows, [rows, zero16 + c])
                            plsc.store_scatter(outb, [rows, zero16 + c],
                                               col * w)
                    return c2
                lax.fori_loop(0, EK // 16, group, 0)

                cps = []
                for g in range(EK // 128):
                    cps.append(pltpu.async_copy(
                        outb.at[pl.ds(g * 128, 128)],
                        acc.at[dstv3_[b].at[g]], sems, add=True))
                for cp in cps:
                    cp.wait()
            return carry
        lax.fori_loop(0, ECH // 2, chunk2, 0)

        plsc.subcore_barrier()
        rows_per = NHALF // 16
        pltpu.sync_copy(
            acc.at[pl.ds(sid * rows_per, rows_per)],
            out.at[cid, pl.ds(nbase + sid * rows_per, rows_per)])
        plsc.subcore_barrier()


def _make_edge_kernel(heads):
    mesh = plsc.VectorSubcoreMesh(core_axis_name="c", subcore_axis_name="s", num_cores=2, num_subcores=16)
    return functools.partial(
        pl.kernel,
        out_type=jax.ShapeDtypeStruct((2, NPAD, 40), jnp.float32),
        mesh=mesh,
        compiler_params=pltpu.CompilerParams(needs_layout_passes=False, use_tc_tiling_on_sc=False),
        scratch_types=(
            [pltpu.VMEM((EK // 128, 128), jnp.int32)] * 6
            + [pltpu.VMEM((EK, 8), jnp.float32)] * 4
            + [pltpu.VMEM((EK, 32), jnp.float32)] * 2
            + [pltpu.VMEM((EK, 40), jnp.float32)]
            + [pltpu.VMEM_SHARED((ACCR, 40), jnp.float32)]
            + [pltpu.SemaphoreType.DMA] * 5
        ),
    )(functools.partial(_edge_body, heads))


def _pool_body(pflat, bidx, zrows, out, rowsb, biv, acc):
    cid = lax.axis_index("c")
    sid = lax.axis_index("s")

    @pl.when(sid == 0)
    def _():
        pltpu.sync_copy(zrows.at[pl.ds(0, NPOOL)], acc)
    plsc.subcore_barrier()

    def chunk(ch, carry):
        rbase = sid * 3200 + ch * 640
        pltpu.sync_copy(pflat.at[pl.ds(cid * NPAD + rbase, 640)], rowsb)
        pltpu.sync_copy(bidx.at[pl.ds(sid * 25 + ch * 5, 5)], biv)
        for g in range(5):
            pltpu.sync_copy(rowsb.at[pl.ds(g * 128, 128)],
                            acc.at[biv.at[g]], add=True)
        return carry
    lax.fori_loop(0, 5, chunk, 0)

    plsc.subcore_barrier()
    sl = pl.ds(sid * (NPOOL // 16), NPOOL // 16)
    pltpu.sync_copy(acc.at[sl], out.at[cid, sl])


def _make_pool_kernel():
    return functools.partial(
        pl.kernel,
        out_type=jax.ShapeDtypeStruct((2, NPOOL, 40), jnp.float32),
        mesh=plsc.VectorSubcoreMesh(core_axis_name="c", subcore_axis_name="s",
                                    num_cores=2, num_subcores=16),
        compiler_params=pltpu.CompilerParams(needs_layout_passes=False, use_tc_tiling_on_sc=False),
        scratch_types=[
            pltpu.VMEM((640, 40), jnp.float32),
            pltpu.VMEM((5, 128), jnp.int32),
            pltpu.VMEM_SHARED((NPOOL, 40), jnp.float32),
        ],
    )(_pool_body)


def _tc_in_body(x_ref, w_ref, aa_ref, ha_ref, hb_ref, sed_ref):
    h = jnp.dot(x_ref[...], w_ref[...], preferred_element_type=jnp.float32)
    sed = jnp.dot(h, aa_ref[...], preferred_element_type=jnp.float32)
    ha_ref[...] = h[:, :32]
    hb_ref[...] = h[:, 32:]
    sed_ref[...] = sed


def _norm_concat(a0, a1, heads_prev):
    if heads_prev == 4:
        return jnp.concatenate([
            a0[:, 0:16] / (a0[:, 32:33] + EPS),
            a0[:, 16:32] / (a0[:, 33:34] + EPS),
            a1[:, 0:16] / (a1[:, 34:35] + EPS),
            a1[:, 16:32] / (a1[:, 35:36] + EPS),
        ], axis=1)
    return jnp.concatenate([
        a0[:, :32] / (a0[:, 32:33] + EPS),
        a1[:, :32] / (a1[:, 32:33] + EPS),
    ], axis=1)


def _epilogue(y, prm_ref, pid):
    y = y + prm_ref[0:1, :]
    y = y * prm_ref[1:2, :] + prm_ref[2:3, :]
    z = jnp.where(y > 0, y, jnp.exp(y) - 1.0)
    rid = pid * BLK + lax.broadcasted_iota(jnp.int32, (BLK, 1), 0)
    return jnp.where(rid < N, z, 0.0), rid


def _tc_mid_body(heads_prev, a0_ref, a1_ref, prm_ref, w_ref, aa_ref,
                 ha_ref, hb_ref, sed_ref):
    y = _norm_concat(a0_ref[...], a1_ref[...], heads_prev)
    z, _ = _epilogue(y, prm_ref, pl.program_id(0))
    h = jnp.dot(z, w_ref[...], preferred_element_type=jnp.float32)
    sed = jnp.dot(h, aa_ref[...], preferred_element_type=jnp.float32)
    ha_ref[...] = h[:, :32]
    hb_ref[...] = h[:, 32:]
    sed_ref[...] = sed


def _tc_fin_body(a0_ref, a1_ref, prm_ref, pa_ref, pb_ref):
    y = _norm_concat(a0_ref[...], a1_ref[...], 1)
    z, rid = _epilogue(y, prm_ref, pl.program_id(0))
    cnt = jnp.where(rid < N, 1.0, 0.0)
    pad = jnp.zeros((BLK, 7), jnp.float32)
    pa_ref[...] = jnp.concatenate([z[:, :32], cnt, pad], axis=1)
    pb_ref[...] = jnp.concatenate([z[:, 32:], cnt, pad], axis=1)


def _tc_div_body(p0_ref, p1_ref, o_ref):
    cnt = jnp.maximum(p0_ref[:, 32:33], 1.0)
    o_ref[...] = jnp.concatenate(
        [p0_ref[:, :32], p1_ref[:, :32]], axis=1) / cnt


def _row_spec(d):
    return pl.BlockSpec((BLK, d), lambda i: (i, 0))


def _full_spec(shape):
    return pl.BlockSpec(shape, lambda i: tuple(0 for _ in shape))


def _tc_in(xp, w, aa):
    return pl.pallas_call(
        _tc_in_body,
        grid=(NPAD // BLK,),
        in_specs=[_row_spec(xp.shape[1]), _full_spec(w.shape),
                  _full_spec(aa.shape)],
        out_specs=[_row_spec(32), _row_spec(32), _row_spec(8)],
        out_shape=[jax.ShapeDtypeStruct((NPAD, 32), jnp.float32),
                   jax.ShapeDtypeStruct((NPAD, 32), jnp.float32),
                   jax.ShapeDtypeStruct((NPAD, 8), jnp.float32)],
    )(xp, w, aa)


def _tc_mid(heads_prev, a0, a1, prm, w, aa):
    return pl.pallas_call(
        functools.partial(_tc_mid_body, heads_prev),
        grid=(NPAD // BLK,),
        in_specs=[_row_spec(40), _row_spec(40), _full_spec(prm.shape),
                  _full_spec(w.shape), _full_spec(aa.shape)],
        out_specs=[_row_spec(32), _row_spec(32), _row_spec(8)],
        out_shape=[jax.ShapeDtypeStruct((NPAD, 32), jnp.float32),
                   jax.ShapeDtypeStruct((NPAD, 32), jnp.float32),
                   jax.ShapeDtypeStruct((NPAD, 8), jnp.float32)],
    )(a0, a1, prm, w, aa)


def _tc_fin(a0, a1, prm):
    return pl.pallas_call(
        _tc_fin_body,
        grid=(NPAD // BLK,),
        in_specs=[_row_spec(40), _row_spec(40), _full_spec(prm.shape)],
        out_specs=[_row_spec(40), _row_spec(40)],
        out_shape=[jax.ShapeDtypeStruct((NPAD, 40), jnp.float32),
                   jax.ShapeDtypeStruct((NPAD, 40), jnp.float32)],
    )(a0, a1, prm)


def _tc_div(p0, p1):
    return pl.pallas_call(
        _tc_div_body,
        grid=(B // BLK,),
        in_specs=[_row_spec(40), _row_spec(40)],
        out_specs=_row_spec(64),
        out_shape=jax.ShapeDtypeStruct((B, 64), jnp.float32),
    )(p0, p1)


def _pack_aa(a_s, a_d):
    heads, c = a_s.shape
    out = jnp.zeros((64, 8), jnp.float32)
    if heads == 4:
        for h in range(4):
            out = out.at[h * c:(h + 1) * c, h].set(a_s[h])
            out = out.at[h * c:(h + 1) * c, 4 + h].set(a_d[h])
    else:
        out = out.at[:, 0].set(a_s[0])
        out = out.at[:, 1].set(a_d[0])
    return out


def _pack_prm(b, g, bb, m, v):
    scale = g / jnp.sqrt(v + 1e-5)
    shift = bb - m * scale
    prm = jnp.zeros((8, 64), jnp.float32)
    return prm.at[0].set(b).at[1].set(scale).at[2].set(shift)


def kernel(x, edge_index, batch, W1, a_src1, a_dst1, b1, bn1_g, bn1_b, bn1_m,
           bn1_v, W2, a_src2, a_dst2, b2, bn2_g, bn2_b, bn2_m, bn2_v, W3,
           a_src3, a_dst3, b3, bn3_g, bn3_b, bn3_m, bn3_v):
    f32 = jnp.float32
    n = x.shape[0]
    loop = jnp.arange(n, dtype=jnp.int32)
    padi = jnp.full((E_PAD - n - edge_index.shape[1],), n, jnp.int32)
    src = jnp.concatenate([edge_index[0].astype(jnp.int32), loop, padi])
    dst = jnp.concatenate([edge_index[1].astype(jnp.int32), loop, padi])
    srcr = src.reshape(E_PAD // 128, 128)
    dstr = dst.reshape(E_PAD // 128, 128)
    zrows = jnp.zeros((NPAD, 40), f32)

    xp = jnp.zeros((NPAD, 16), f32).at[:n, :x.shape[1]].set(x)
    w1p = jnp.zeros((16, 64), f32).at[:W1.shape[0]].set(W1)

    ha, hb, sed = _tc_in(xp, w1p, _pack_aa(a_src1, a_dst1))
    edge1 = _make_edge_kernel(4)
    a = edge1(srcr, dstr, jnp.concatenate([ha, hb]), sed, zrows)

    prm1 = _pack_prm(b1, bn1_g, bn1_b, bn1_m, bn1_v)
    ha, hb, sed = _tc_mid(4, a[0], a[1], prm1, W2, _pack_aa(a_src2, a_dst2))
    a = edge1(srcr, dstr, jnp.concatenate([ha, hb]), sed, zrows)

    prm2 = _pack_prm(b2, bn2_g, bn2_b, bn2_m, bn2_v)
    ha, hb, sed = _tc_mid(4, a[0], a[1], prm2, W3, _pack_aa(a_src3, a_dst3))
    edge3 = _make_edge_kernel(1)
    a = edge3(srcr, dstr, jnp.concatenate([ha, hb]), sed, zrows)

    prm3 = _pack_prm(b3, bn3_g, bn3_b, bn3_m, bn3_v)
    pa, pb = _tc_fin(a[0], a[1], prm3)

    bpad = jnp.concatenate(
        [batch.astype(jnp.int32), jnp.full((NPAD - n,), B, jnp.int32)])
    p = _make_pool_kernel()(jnp.concatenate([pa, pb]),
                     bpad.reshape(NPAD // 128, 128), zrows)
    return _tc_div(p[0, :B], p[1, :B])


# speedup vs baseline: 34.3859x; 1.5859x over previous
import functools
import jax
import jax.numpy as jnp
from jax import lax
from jax.experimental import pallas as pl
from jax.experimental.pallas import tpu as pltpu
from jax.experimental.pallas import tpu_sc as plsc

N = 50000
B = 1024
NPAD = 51200
NPOOL = 1152
EK = 256
ECH = 208
EPT = EK * ECH
E_PAD = EPT * 16
NHALF = 25600
ACCR = NHALF + 8
BLK = 512
EPS = 1e-16


def _edge_body(heads, srcr, dstr, hflat, sed, zrows, out,
               srcv, dstv, srcv2a, srcv2b, dstv3a, dstv3b,
               sedda, seddb, hrowsa, hrowsb, outb, acc,
               semi, sem1, sem2, sems):
    cid = lax.axis_index("c")
    sid = lax.axis_index("s")
    lanes = lax.iota(jnp.int32, 16)
    zero16 = lanes * 0
    srcv2_ = (srcv2a, srcv2b)
    dstv3_ = (dstv3a, dstv3b)
    sedd_ = (sedda, seddb)
    hrows_ = (hrowsa, hrowsb)

    pltpu.sync_copy(zrows.at[pl.ds(0, EK)], outb)

    for p in range(NPAD // NHALF):
        nbase = p * NHALF

        @pl.when(sid == 0)
        def _():
            pltpu.sync_copy(zrows.at[pl.ds(0, ACCR)], acc)
        plsc.subcore_barrier()

        def issue(ch, b):
            rb = sid * (EPT // 128) + ch * (EK // 128)
            c1 = pltpu.async_copy(srcr.at[pl.ds(rb, EK // 128)], srcv, semi)
            c2 = pltpu.async_copy(dstr.at[pl.ds(rb, EK // 128)], dstv, semi)
            c1.wait()
            c2.wait()
            for i in range(EK // 128):
                for j in range(8):
                    sl16 = pl.ds(j * 16, 16)
                    srcv2_[b][i, sl16] = srcv[i, sl16] + cid * NPAD
                    dl = dstv[i, sl16] - nbase
                    ok = (dl >= 0) & (dl < NHALF)
                    dstv3_[b][i, sl16] = jnp.where(ok, dl, NHALF)
            for g in range(EK // 128):
                sl = pl.ds(g * 128, 128)
                pltpu.async_copy(sed.at[dstv.at[g]], sedd_[b].at[sl], sem1)
                pltpu.async_copy(hflat.at[srcv2_[b].at[g]],
                                 hrows_[b].at[sl], sem2)

        issue(0, 0)

        def chunk2(ii, carry):
            for b in range(2):
                ch = ii * 2 + b
                pltpu.make_async_copy(
                    sed.at[pl.ds(0, EK)], sedd_[b], sem1).wait()
                pltpu.make_async_copy(
                    hflat.at[pl.ds(0, EK)], hrows_[b], sem2).wait()

                @pl.when(ch + 1 < ECH)
                def _():
                    issue(ch + 1, 1 - b)

                sedd = sedd_[b]
                hrows = hrows_[b]

                def group(g, c2):
                    rows = g * 16 + lanes
                    if heads == 4:
                        ws = []
                        for h in range(4):
                            es = plsc.load_gather(hrows, [rows, zero16 + 32 + h])
                            ed = plsc.load_gather(sedd, [rows, zero16 + h])
                            e = es + ed
                            e = jnp.where(e >= 0, e, e * jnp.float32(0.2))
                            ws.append(jnp.exp(e))
                        c0 = cid == 0
                        wlo = jnp.where(c0, ws[0], ws[2])
                        whi = jnp.where(c0, ws[1], ws[3])
                        for h in range(4):
                            plsc.store_scatter(outb, [rows, zero16 + 32 + h],
                                               ws[h])
                        for c in range(32):
                            col = plsc.load_gather(hrows, [rows, zero16 + c])
                            w = wlo if c < 16 else whi
                            plsc.store_scatter(outb, [rows, zero16 + c],
                                               col * w)
                    else:
                        es = plsc.load_gather(hrows, [rows, zero16 + 32])
                        ed = plsc.load_gather(sedd, [rows, zero16])
                        e = es + ed
                        e = jnp.where(e >= 0, e, e * jnp.float32(0.2))
                        w = jnp.exp(e)
                        plsc.store_scatter(outb, [rows, zero16 + 32], w)
                        for c in range(32):
                            col = plsc.load_gather(hrows, [rows, zero16 + c])
                            plsc.store_scatter(outb, [rows, zero16 + c],
                                               col * w)
                    return c2
                lax.fori_loop(0, EK // 16, group, 0)

                cps = []
                for g in range(EK // 128):
                    cps.append(pltpu.async_copy(
                        outb.at[pl.ds(g * 128, 128)],
                        acc.at[dstv3_[b].at[g]], sems, add=True))
                for cp in cps:
                    cp.wait()
            return carry
        lax.fori_loop(0, ECH // 2, chunk2, 0)

        plsc.subcore_barrier()
        rows_per = NHALF // 16
        pltpu.sync_copy(
            acc.at[pl.ds(sid * rows_per, rows_per)],
            out.at[cid, pl.ds(nbase + sid * rows_per, rows_per)])
        plsc.subcore_barrier()


def _make_edge_kernel(heads):
    mesh = plsc.VectorSubcoreMesh(core_axis_name="c", subcore_axis_name="s", num_cores=2, num_subcores=16)
    return functools.partial(
        pl.kernel,
        out_type=jax.ShapeDtypeStruct((2, NPAD, 40), jnp.float32),
        mesh=mesh,
        compiler_params=pltpu.CompilerParams(needs_layout_passes=False, use_tc_tiling_on_sc=False),
        scratch_types=(
            [pltpu.VMEM((EK // 128, 128), jnp.int32)] * 6
            + [pltpu.VMEM((EK, 8), jnp.float32)] * 2
            + [pltpu.VMEM((EK, 40), jnp.float32)] * 2
            + [pltpu.VMEM((EK, 40), jnp.float32)]
            + [pltpu.VMEM_SHARED((ACCR, 40), jnp.float32)]
            + [pltpu.SemaphoreType.DMA] * 4
        ),
    )(functools.partial(_edge_body, heads))


def _pool_body(pflat, bidx, zrows, out, rowsb, biv, acc):
    cid = lax.axis_index("c")
    sid = lax.axis_index("s")

    @pl.when(sid == 0)
    def _():
        pltpu.sync_copy(zrows.at[pl.ds(0, NPOOL)], acc)
    plsc.subcore_barrier()

    def chunk(ch, carry):
        rbase = sid * 3200 + ch * 640
        pltpu.sync_copy(pflat.at[pl.ds(cid * NPAD + rbase, 640)], rowsb)
        pltpu.sync_copy(bidx.at[pl.ds(sid * 25 + ch * 5, 5)], biv)
        for g in range(5):
            pltpu.sync_copy(rowsb.at[pl.ds(g * 128, 128)],
                            acc.at[biv.at[g]], add=True)
        return carry
    lax.fori_loop(0, 5, chunk, 0)

    plsc.subcore_barrier()
    sl = pl.ds(sid * (NPOOL // 16), NPOOL // 16)
    pltpu.sync_copy(acc.at[sl], out.at[cid, sl])


def _make_pool_kernel():
    return functools.partial(
        pl.kernel,
        out_type=jax.ShapeDtypeStruct((2, NPOOL, 40), jnp.float32),
        mesh=plsc.VectorSubcoreMesh(core_axis_name="c", subcore_axis_name="s",
                                    num_cores=2, num_subcores=16),
        compiler_params=pltpu.CompilerParams(needs_layout_passes=False, use_tc_tiling_on_sc=False),
        scratch_types=[
            pltpu.VMEM((640, 40), jnp.float32),
            pltpu.VMEM((5, 128), jnp.int32),
            pltpu.VMEM_SHARED((NPOOL, 40), jnp.float32),
        ],
    )(_pool_body)


def _emit_tables(h, sed, ha_ref, hb_ref, ed_ref):
    z4 = jnp.zeros((BLK, 4), jnp.float32)
    es = sed[:, :4]
    ha_ref[...] = jnp.concatenate([h[:, :32], es, z4], axis=1)
    hb_ref[...] = jnp.concatenate([h[:, 32:], es, z4], axis=1)
    ed_ref[...] = jnp.concatenate([sed[:, 4:], z4], axis=1)


def _tc_in_body(x_ref, w_ref, aa_ref, ha_ref, hb_ref, ed_ref):
    h = jnp.dot(x_ref[...], w_ref[...], preferred_element_type=jnp.float32)
    sed = jnp.dot(h, aa_ref[...], preferred_element_type=jnp.float32)
    _emit_tables(h, sed, ha_ref, hb_ref, ed_ref)


def _norm_concat(a0, a1, heads_prev):
    if heads_prev == 4:
        return jnp.concatenate([
            a0[:, 0:16] / (a0[:, 32:33] + EPS),
            a0[:, 16:32] / (a0[:, 33:34] + EPS),
            a1[:, 0:16] / (a1[:, 34:35] + EPS),
            a1[:, 16:32] / (a1[:, 35:36] + EPS),
        ], axis=1)
    return jnp.concatenate([
        a0[:, :32] / (a0[:, 32:33] + EPS),
        a1[:, :32] / (a1[:, 32:33] + EPS),
    ], axis=1)


def _epilogue(y, prm_ref, pid):
    y = y + prm_ref[0:1, :]
    y = y * prm_ref[1:2, :] + prm_ref[2:3, :]
    z = jnp.where(y > 0, y, jnp.exp(y) - 1.0)
    rid = pid * BLK + lax.broadcasted_iota(jnp.int32, (BLK, 1), 0)
    return jnp.where(rid < N, z, 0.0), rid


def _tc_mid_body(heads_prev, a0_ref, a1_ref, prm_ref, w_ref, aa_ref,
                 ha_ref, hb_ref, ed_ref):
    y = _norm_concat(a0_ref[...], a1_ref[...], heads_prev)
    z, _ = _epilogue(y, prm_ref, pl.program_id(0))
    h = jnp.dot(z, w_ref[...], preferred_element_type=jnp.float32)
    sed = jnp.dot(h, aa_ref[...], preferred_element_type=jnp.float32)
    _emit_tables(h, sed, ha_ref, hb_ref, ed_ref)


def _tc_fin_body(a0_ref, a1_ref, prm_ref, pa_ref, pb_ref):
    y = _norm_concat(a0_ref[...], a1_ref[...], 1)
    z, rid = _epilogue(y, prm_ref, pl.program_id(0))
    cnt = jnp.where(rid < N, 1.0, 0.0)
    pad = jnp.zeros((BLK, 7), jnp.float32)
    pa_ref[...] = jnp.concatenate([z[:, :32], cnt, pad], axis=1)
    pb_ref[...] = jnp.concatenate([z[:, 32:], cnt, pad], axis=1)


def _tc_div_body(p0_ref, p1_ref, o_ref):
    cnt = jnp.maximum(p0_ref[:, 32:33], 1.0)
    o_ref[...] = jnp.concatenate(
        [p0_ref[:, :32], p1_ref[:, :32]], axis=1) / cnt


def _row_spec(d):
    return pl.BlockSpec((BLK, d), lambda i: (i, 0))


def _full_spec(shape):
    return pl.BlockSpec(shape, lambda i: tuple(0 for _ in shape))


def _tc_in(xp, w, aa):
    return pl.pallas_call(
        _tc_in_body,
        grid=(NPAD // BLK,),
        in_specs=[_row_spec(xp.shape[1]), _full_spec(w.shape),
                  _full_spec(aa.shape)],
        out_specs=[_row_spec(40), _row_spec(40), _row_spec(8)],
        out_shape=[jax.ShapeDtypeStruct((NPAD, 40), jnp.float32),
                   jax.ShapeDtypeStruct((NPAD, 40), jnp.float32),
                   jax.ShapeDtypeStruct((NPAD, 8), jnp.float32)],
    )(xp, w, aa)


def _tc_mid(heads_prev, a0, a1, prm, w, aa):
    return pl.pallas_call(
        functools.partial(_tc_mid_body, heads_prev),
        grid=(NPAD // BLK,),
        in_specs=[_row_spec(40), _row_spec(40), _full_spec(prm.shape),
                  _full_spec(w.shape), _full_spec(aa.shape)],
        out_specs=[_row_spec(40), _row_spec(40), _row_spec(8)],
        out_shape=[jax.ShapeDtypeStruct((NPAD, 40), jnp.float32),
                   jax.ShapeDtypeStruct((NPAD, 40), jnp.float32),
                   jax.ShapeDtypeStruct((NPAD, 8), jnp.float32)],
    )(a0, a1, prm, w, aa)


def _tc_fin(a0, a1, prm):
    return pl.pallas_call(
        _tc_fin_body,
        grid=(NPAD // BLK,),
        in_specs=[_row_spec(40), _row_spec(40), _full_spec(prm.shape)],
        out_specs=[_row_spec(40), _row_spec(40)],
        out_shape=[jax.ShapeDtypeStruct((NPAD, 40), jnp.float32),
                   jax.ShapeDtypeStruct((NPAD, 40), jnp.float32)],
    )(a0, a1, prm)


def _tc_div(p0, p1):
    return pl.pallas_call(
        _tc_div_body,
        grid=(B // BLK,),
        in_specs=[_row_spec(40), _row_spec(40)],
        out_specs=_row_spec(64),
        out_shape=jax.ShapeDtypeStruct((B, 64), jnp.float32),
    )(p0, p1)


def _pack_aa(a_s, a_d):
    heads, c = a_s.shape
    out = jnp.zeros((64, 8), jnp.float32)
    if heads == 4:
        for h in range(4):
            out = out.at[h * c:(h + 1) * c, h].set(a_s[h])
            out = out.at[h * c:(h + 1) * c, 4 + h].set(a_d[h])
    else:
        out = out.at[:, 0].set(a_s[0])
        out = out.at[:, 4].set(a_d[0])
    return out


def _pack_prm(b, g, bb, m, v):
    scale = g / jnp.sqrt(v + 1e-5)
    shift = bb - m * scale
    prm = jnp.zeros((8, 64), jnp.float32)
    return prm.at[0].set(b).at[1].set(scale).at[2].set(shift)


def kernel(x, edge_index, batch, W1, a_src1, a_dst1, b1, bn1_g, bn1_b, bn1_m,
           bn1_v, W2, a_src2, a_dst2, b2, bn2_g, bn2_b, bn2_m, bn2_v, W3,
           a_src3, a_dst3, b3, bn3_g, bn3_b, bn3_m, bn3_v):
    f32 = jnp.float32
    n = x.shape[0]
    loop = jnp.arange(n, dtype=jnp.int32)
    padi = jnp.full((E_PAD - n - edge_index.shape[1],), n, jnp.int32)
    src = jnp.concatenate([edge_index[0].astype(jnp.int32), loop, padi])
    dst = jnp.concatenate([edge_index[1].astype(jnp.int32), loop, padi])
    srcr = src.reshape(E_PAD // 128, 128)
    dstr = dst.reshape(E_PAD // 128, 128)
    zrows = jnp.zeros((NPAD, 40), f32)

    xp = jnp.zeros((NPAD, 16), f32).at[:n, :x.shape[1]].set(x)
    w1p = jnp.zeros((16, 64), f32).at[:W1.shape[0]].set(W1)

    ha, hb, sed = _tc_in(xp, w1p, _pack_aa(a_src1, a_dst1))
    edge1 = _make_edge_kernel(4)
    a = edge1(srcr, dstr, jnp.concatenate([ha, hb]), sed, zrows)

    prm1 = _pack_prm(b1, bn1_g, bn1_b, bn1_m, bn1_v)
    ha, hb, sed = _tc_mid(4, a[0], a[1], prm1, W2, _pack_aa(a_src2, a_dst2))
    a = edge1(srcr, dstr, jnp.concatenate([ha, hb]), sed, zrows)

    prm2 = _pack_prm(b2, bn2_g, bn2_b, bn2_m, bn2_v)
    ha, hb, sed = _tc_mid(4, a[0], a[1], prm2, W3, _pack_aa(a_src3, a_dst3))
    edge3 = _make_edge_kernel(1)
    a = edge3(srcr, dstr, jnp.concatenate([ha, hb]), sed, zrows)

    prm3 = _pack_prm(b3, bn3_g, bn3_b, bn3_m, bn3_v)
    pa, pb = _tc_fin(a[0], a[1], prm3)

    bpad = jnp.concatenate(
        [batch.astype(jnp.int32), jnp.full((NPAD - n,), B, jnp.int32)])
    p = _make_pool_kernel()(jnp.concatenate([pa, pb]),
                     bpad.reshape(NPAD // 128, 128), zrows)
    return _tc_div(p[0, :B], p[1, :B])


# speedup vs baseline: 34.5084x; 1.0036x over previous
import functools
import jax
import jax.numpy as jnp
from jax import lax
from jax.experimental import pallas as pl
from jax.experimental.pallas import tpu as pltpu
from jax.experimental.pallas import tpu_sc as plsc

N = 50000
B = 1024
NPAD = 51200
NPOOL = 1152
EK = 256
ECH = 208
EPT = EK * ECH
E_PAD = EPT * 16
NHALF = 25600
ACCR = NHALF + 256
BLK = 512
EPS = 1e-16


def _edge_body(heads, srcr, dstr, hflat, sed, zrows, out,
               srcv, dstv, srcv2a, srcv2b, dstv3a, dstv3b,
               sedda, seddb, hrowsa, hrowsb, outb, acc,
               semi, sem1, sem2, sems):
    cid = lax.axis_index("c")
    sid = lax.axis_index("s")
    lanes = lax.iota(jnp.int32, 16)
    zero16 = lanes * 0
    srcv2_ = (srcv2a, srcv2b)
    dstv3_ = (dstv3a, dstv3b)
    sedd_ = (sedda, seddb)
    hrows_ = (hrowsa, hrowsb)

    pltpu.sync_copy(zrows.at[pl.ds(0, EK)], outb)

    for p in range(NPAD // NHALF):
        nbase = p * NHALF

        @pl.when(sid == 0)
        def _():
            pltpu.sync_copy(zrows.at[pl.ds(0, ACCR)], acc)
        plsc.subcore_barrier()

        def issue(ch, b):
            rb = sid * (EPT // 128) + ch * (EK // 128)
            c1 = pltpu.async_copy(srcr.at[pl.ds(rb, EK // 128)], srcv, semi)
            c2 = pltpu.async_copy(dstr.at[pl.ds(rb, EK // 128)], dstv, semi)
            c1.wait()
            c2.wait()
            for i in range(EK // 128):
                for j in range(8):
                    sl16 = pl.ds(j * 16, 16)
                    srcv2_[b][i, sl16] = srcv[i, sl16] + cid * NPAD
                    dv = dstv[i, sl16]
                    dl = dv - nbase
                    ok = (dl >= 0) & (dl < NHALF)
                    dstv3_[b][i, sl16] = jnp.where(ok, dl, NHALF + (dv & 255))
            for g in range(EK // 128):
                sl = pl.ds(g * 128, 128)
                pltpu.async_copy(sed.at[dstv.at[g]], sedd_[b].at[sl], sem1)
                pltpu.async_copy(hflat.at[srcv2_[b].at[g]],
                                 hrows_[b].at[sl], sem2)

        issue(0, 0)

        def chunk2(ii, carry):
            for b in range(2):
                ch = ii * 2 + b
                pltpu.make_async_copy(
                    sed.at[pl.ds(0, EK)], sedd_[b], sem1).wait()
                pltpu.make_async_copy(
                    hflat.at[pl.ds(0, EK)], hrows_[b], sem2).wait()

                @pl.when(ch + 1 < ECH)
                def _():
                    issue(ch + 1, 1 - b)

                sedd = sedd_[b]
                hrows = hrows_[b]

                def group(g, c2):
                    rows = g * 16 + lanes
                    if heads == 4:
                        ws = []
                        for h in range(4):
                            es = plsc.load_gather(hrows, [rows, zero16 + 32 + h])
                            ed = plsc.load_gather(sedd, [rows, zero16 + h])
                            e = es + ed
                            e = jnp.where(e >= 0, e, e * jnp.float32(0.2))
                            ws.append(jnp.exp(e))
                        c0 = cid == 0
                        wlo = jnp.where(c0, ws[0], ws[2])
                        whi = jnp.where(c0, ws[1], ws[3])
                        for h in range(4):
                            plsc.store_scatter(outb, [rows, zero16 + 32 + h],
                                               ws[h])
                        for c in range(32):
                            col = plsc.load_gather(hrows, [rows, zero16 + c])
                            w = wlo if c < 16 else whi
                            plsc.store_scatter(outb, [rows, zero16 + c],
                                               col * w)
                    else:
                        es = plsc.load_gather(hrows, [rows, zero16 + 32])
                        ed = plsc.load_gather(sedd, [rows, zero16])
                        e = es + ed
                        e = jnp.where(e >= 0, e, e * jnp.float32(0.2))
                        w = jnp.exp(e)
                        plsc.store_scatter(outb, [rows, zero16 + 32], w)
                        for c in range(32):
                            col = plsc.load_gather(hrows, [rows, zero16 + c])
                            plsc.store_scatter(outb, [rows, zero16 + c],
                                               col * w)
                    return c2
                lax.fori_loop(0, EK // 16, group, 0)

                cps = []
                for g in range(EK // 128):
                    cps.append(pltpu.async_copy(
                        outb.at[pl.ds(g * 128, 128)],
                        acc.at[dstv3_[b].at[g]], sems, add=True))
                for cp in cps:
                    cp.wait()
            return carry
        lax.fori_loop(0, ECH // 2, chunk2, 0)

        plsc.subcore_barrier()
        rows_per = NHALF // 16
        pltpu.sync_copy(
            acc.at[pl.ds(sid * rows_per, rows_per)],
            out.at[cid, pl.ds(nbase + sid * rows_per, rows_per)])
        plsc.subcore_barrier()


def _make_edge_kernel(heads):
    mesh = plsc.VectorSubcoreMesh(core_axis_name="c", subcore_axis_name="s", num_cores=2, num_subcores=16)
    return functools.partial(
        pl.kernel,
        out_type=jax.ShapeDtypeStruct((2, NPAD, 40), jnp.float32),
        mesh=mesh,
        compiler_params=pltpu.CompilerParams(needs_layout_passes=False, use_tc_tiling_on_sc=False),
        scratch_types=(
            [pltpu.VMEM((EK // 128, 128), jnp.int32)] * 6
            + [pltpu.VMEM((EK, 8), jnp.float32)] * 2
            + [pltpu.VMEM((EK, 40), jnp.float32)] * 2
            + [pltpu.VMEM((EK, 40), jnp.float32)]
            + [pltpu.VMEM_SHARED((ACCR, 40), jnp.float32)]
            + [pltpu.SemaphoreType.DMA] * 4
        ),
    )(functools.partial(_edge_body, heads))


def _pool_body(pflat, bidx, zrows, out, rowsb, biv, acc):
    cid = lax.axis_index("c")
    sid = lax.axis_index("s")

    @pl.when(sid == 0)
    def _():
        pltpu.sync_copy(zrows.at[pl.ds(0, NPOOL)], acc)
    plsc.subcore_barrier()

    def chunk(ch, carry):
        rbase = sid * 3200 + ch * 640
        pltpu.sync_copy(pflat.at[pl.ds(cid * NPAD + rbase, 640)], rowsb)
        pltpu.sync_copy(bidx.at[pl.ds(sid * 25 + ch * 5, 5)], biv)
        for g in range(5):
            pltpu.sync_copy(rowsb.at[pl.ds(g * 128, 128)],
                            acc.at[biv.at[g]], add=True)
        return carry
    lax.fori_loop(0, 5, chunk, 0)

    plsc.subcore_barrier()
    sl = pl.ds(sid * (NPOOL // 16), NPOOL // 16)
    pltpu.sync_copy(acc.at[sl], out.at[cid, sl])


def _make_pool_kernel():
    return functools.partial(
        pl.kernel,
        out_type=jax.ShapeDtypeStruct((2, NPOOL, 40), jnp.float32),
        mesh=plsc.VectorSubcoreMesh(core_axis_name="c", subcore_axis_name="s",
                                    num_cores=2, num_subcores=16),
        compiler_params=pltpu.CompilerParams(needs_layout_passes=False, use_tc_tiling_on_sc=False),
        scratch_types=[
            pltpu.VMEM((640, 40), jnp.float32),
            pltpu.VMEM((5, 128), jnp.int32),
            pltpu.VMEM_SHARED((NPOOL, 40), jnp.float32),
        ],
    )(_pool_body)


def _emit_tables(h, sed, ha_ref, hb_ref, ed_ref):
    z4 = jnp.zeros((BLK, 4), jnp.float32)
    es = sed[:, :4]
    ha_ref[...] = jnp.concatenate([h[:, :32], es, z4], axis=1)
    hb_ref[...] = jnp.concatenate([h[:, 32:], es, z4], axis=1)
    ed_ref[...] = jnp.concatenate([sed[:, 4:], z4], axis=1)


def _tc_in_body(x_ref, w_ref, aa_ref, ha_ref, hb_ref, ed_ref):
    h = jnp.dot(x_ref[...], w_ref[...], preferred_element_type=jnp.float32)
    sed = jnp.dot(h, aa_ref[...], preferred_element_type=jnp.float32)
    _emit_tables(h, sed, ha_ref, hb_ref, ed_ref)


def _norm_concat(a0, a1, heads_prev):
    if heads_prev == 4:
        return jnp.concatenate([
            a0[:, 0:16] / (a0[:, 32:33] + EPS),
            a0[:, 16:32] / (a0[:, 33:34] + EPS),
            a1[:, 0:16] / (a1[:, 34:35] + EPS),
            a1[:, 16:32] / (a1[:, 35:36] + EPS),
        ], axis=1)
    return jnp.concatenate([
        a0[:, :32] / (a0[:, 32:33] + EPS),
        a1[:, :32] / (a1[:, 32:33] + EPS),
    ], axis=1)


def _epilogue(y, prm_ref, pid):
    y = y + prm_ref[0:1, :]
    y = y * prm_ref[1:2, :] + prm_ref[2:3, :]
    z = jnp.where(y > 0, y, jnp.exp(y) - 1.0)
    rid = pid * BLK + lax.broadcasted_iota(jnp.int32, (BLK, 1), 0)
    return jnp.where(rid < N, z, 0.0), rid


def _tc_mid_body(heads_prev, a0_ref, a1_ref, prm_ref, w_ref, aa_ref,
                 ha_ref, hb_ref, ed_ref):
    y = _norm_concat(a0_ref[...], a1_ref[...], heads_prev)
    z, _ = _epilogue(y, prm_ref, pl.program_id(0))
    h = jnp.dot(z, w_ref[...], preferred_element_type=jnp.float32)
    sed = jnp.dot(h, aa_ref[...], preferred_element_type=jnp.float32)
    _emit_tables(h, sed, ha_ref, hb_ref, ed_ref)


def _tc_fin_body(a0_ref, a1_ref, prm_ref, pa_ref, pb_ref):
    y = _norm_concat(a0_ref[...], a1_ref[...], 1)
    z, rid = _epilogue(y, prm_ref, pl.program_id(0))
    cnt = jnp.where(rid < N, 1.0, 0.0)
    pad = jnp.zeros((BLK, 7), jnp.float32)
    pa_ref[...] = jnp.concatenate([z[:, :32], cnt, pad], axis=1)
    pb_ref[...] = jnp.concatenate([z[:, 32:], cnt, pad], axis=1)


def _tc_div_body(p0_ref, p1_ref, o_ref):
    cnt = jnp.maximum(p0_ref[:, 32:33], 1.0)
    o_ref[...] = jnp.concatenate(
        [p0_ref[:, :32], p1_ref[:, :32]], axis=1) / cnt


def _row_spec(d):
    return pl.BlockSpec((BLK, d), lambda i: (i, 0))


def _full_spec(shape):
    return pl.BlockSpec(shape, lambda i: tuple(0 for _ in shape))


def _tc_in(xp, w, aa):
    return pl.pallas_call(
        _tc_in_body,
        grid=(NPAD // BLK,),
        in_specs=[_row_spec(xp.shape[1]), _full_spec(w.shape),
                  _full_spec(aa.shape)],
        out_specs=[_row_spec(40), _row_spec(40), _row_spec(8)],
        out_shape=[jax.ShapeDtypeStruct((NPAD, 40), jnp.float32),
                   jax.ShapeDtypeStruct((NPAD, 40), jnp.float32),
                   jax.ShapeDtypeStruct((NPAD, 8), jnp.float32)],
    )(xp, w, aa)


def _tc_mid(heads_prev, a0, a1, prm, w, aa):
    return pl.pallas_call(
        functools.partial(_tc_mid_body, heads_prev),
        grid=(NPAD // BLK,),
        in_specs=[_row_spec(40), _row_spec(40), _full_spec(prm.shape),
                  _full_spec(w.shape), _full_spec(aa.shape)],
        out_specs=[_row_spec(40), _row_spec(40), _row_spec(8)],
        out_shape=[jax.ShapeDtypeStruct((NPAD, 40), jnp.float32),
                   jax.ShapeDtypeStruct((NPAD, 40), jnp.float32),
                   jax.ShapeDtypeStruct((NPAD, 8), jnp.float32)],
    )(a0, a1, prm, w, aa)


def _tc_fin(a0, a1, prm):
    return pl.pallas_call(
        _tc_fin_body,
        grid=(NPAD // BLK,),
        in_specs=[_row_spec(40), _row_spec(40), _full_spec(prm.shape)],
        out_specs=[_row_spec(40), _row_spec(40)],
        out_shape=[jax.ShapeDtypeStruct((NPAD, 40), jnp.float32),
                   jax.ShapeDtypeStruct((NPAD, 40), jnp.float32)],
    )(a0, a1, prm)


def _tc_div(p0, p1):
    return pl.pallas_call(
        _tc_div_body,
        grid=(B // BLK,),
        in_specs=[_row_spec(40), _row_spec(40)],
        out_specs=_row_spec(64),
        out_shape=jax.ShapeDtypeStruct((B, 64), jnp.float32),
    )(p0, p1)


def _pack_aa(a_s, a_d):
    heads, c = a_s.shape
    out = jnp.zeros((64, 8), jnp.float32)
    if heads == 4:
        for h in range(4):
            out = out.at[h * c:(h + 1) * c, h].set(a_s[h])
            out = out.at[h * c:(h + 1) * c, 4 + h].set(a_d[h])
    else:
        out = out.at[:, 0].set(a_s[0])
        out = out.at[:, 4].set(a_d[0])
    return out


def _pack_prm(b, g, bb, m, v):
    scale = g / jnp.sqrt(v + 1e-5)
    shift = bb - m * scale
    prm = jnp.zeros((8, 64), jnp.float32)
    return prm.at[0].set(b).at[1].set(scale).at[2].set(shift)


def kernel(x, edge_index, batch, W1, a_src1, a_dst1, b1, bn1_g, bn1_b, bn1_m,
           bn1_v, W2, a_src2, a_dst2, b2, bn2_g, bn2_b, bn2_m, bn2_v, W3,
           a_src3, a_dst3, b3, bn3_g, bn3_b, bn3_m, bn3_v):
    f32 = jnp.float32
    n = x.shape[0]
    loop = jnp.arange(n, dtype=jnp.int32)
    padi = jnp.full((E_PAD - n - edge_index.shape[1],), n, jnp.int32)
    src = jnp.concatenate([edge_index[0].astype(jnp.int32), loop, padi])
    dst = jnp.concatenate([edge_index[1].astype(jnp.int32), loop, padi])
    srcr = src.reshape(E_PAD // 128, 128)
    dstr = dst.reshape(E_PAD // 128, 128)
    zrows = jnp.zeros((NPAD, 40), f32)

    xp = jnp.zeros((NPAD, 16), f32).at[:n, :x.shape[1]].set(x)
    w1p = jnp.zeros((16, 64), f32).at[:W1.shape[0]].set(W1)

    ha, hb, sed = _tc_in(xp, w1p, _pack_aa(a_src1, a_dst1))
    edge1 = _make_edge_kernel(4)
    a = edge1(srcr, dstr, jnp.concatenate([ha, hb]), sed, zrows)

    prm1 = _pack_prm(b1, bn1_g, bn1_b, bn1_m, bn1_v)
    ha, hb, sed = _tc_mid(4, a[0], a[1], prm1, W2, _pack_aa(a_src2, a_dst2))
    a = edge1(srcr, dstr, jnp.concatenate([ha, hb]), sed, zrows)

    prm2 = _pack_prm(b2, bn2_g, bn2_b, bn2_m, bn2_v)
    ha, hb, sed = _tc_mid(4, a[0], a[1], prm2, W3, _pack_aa(a_src3, a_dst3))
    edge3 = _make_edge_kernel(1)
    a = edge3(srcr, dstr, jnp.concatenate([ha, hb]), sed, zrows)

    prm3 = _pack_prm(b3, bn3_g, bn3_b, bn3_m, bn3_v)
    pa, pb = _tc_fin(a[0], a[1], prm3)

    bpad = jnp.concatenate(
        [batch.astype(jnp.int32), jnp.full((NPAD - n,), B, jnp.int32)])
    p = _make_pool_kernel()(jnp.concatenate([pa, pb]),
                     bpad.reshape(NPAD // 128, 128), zrows)
    return _tc_div(p[0, :B], p[1, :B])
